# baseline (speedup 1.0000x reference)
import functools

import jax
import jax.numpy as jnp
from jax import lax
from jax.experimental import pallas as pl
from jax.experimental.pallas import tpu as pltpu
from jax.experimental.pallas import tpu_sc as plsc

N_ = 10000
E_ = 160000
H_ = 8
DH_ = 32
BL = 1000
GRID = N_ // BL
EB = 80
NB = E_ // 16 // EB
RPW = N_ // 16


def _make_sck(G, slope):
    GW = 256 // G
    GL = max(1, 128 // GW)
    CW = 128 // GL
    mesh = plsc.VectorSubcoreMesh(core_axis_name="c", subcore_axis_name="s")

    @functools.partial(
        pl.kernel,
        out_type=jax.ShapeDtypeStruct((2 * N_, 144), jnp.float32),
        mesh=mesh,
        compiler_params=pltpu.CompilerParams(use_tc_tiling_on_sc=False,
                                             needs_layout_passes=False),
        scratch_types=(
            [pltpu.VMEM((EB,), jnp.int32)] * 12
            + [pltpu.VMEM((EB, 144), jnp.float32)] * 3
            + [pltpu.VMEM((EB, 16), jnp.float32)] * 3
            + [
                pltpu.VMEM((16, EB), jnp.float32),
                pltpu.VMEM_SHARED((N_, 144), jnp.float32),
            ]
            + [pltpu.SemaphoreType.DMA] * 7
        ),
    )
    def sck(tab, er, src2, dst2, out, si0, si1, si2, si3, si4, si5,
            di0, di1, di2, di3, di4, di5, gb0, gb1, gb2, eb0, eb1, eb2,
            exb, acc, semi, sg0, sg1, sg2, ss0, ss1, ss2):
        c = lax.axis_index("c")
        s = lax.axis_index("s")
        zero16 = jnp.zeros((16,), jnp.float32)
        sib = (si0, si1, si2, si3, si4, si5)
        dib = (di0, di1, di2, di3, di4, di5)
        gbuf = (gb0, gb1, gb2)
        erb = (eb0, eb1, eb2)
        sg = (sg0, sg1, sg2)
        ss = (ss0, ss1, ss2)

        def zrow(e, carry):
            for cc in range(9):
                gb0[e, pl.ds(cc * 16, 16)] = zero16
            return carry
        lax.fori_loop(0, EB, zrow, 0)
        NCH = N_ // EB
        for jj in range((NCH + 15) // 16):
            j = s + jj * 16

            @pl.when(j < NCH)
            def _():
                off = pl.multiple_of(j * EB, 8)
                pltpu.sync_copy(gb0, acc.at[pl.ds(off, EB)])

        cN = c * N_
        plsc.subcore_barrier()

        def stage_idx(b, islot):
            boff = pl.multiple_of(b * EB, 8)
            pltpu.async_copy(src2.at[s, pl.ds(boff, EB)], sib[islot], semi)
            pltpu.async_copy(dst2.at[s, pl.ds(boff, EB)], dib[islot], semi)

        def issue_gathers(gslot, islot):
            pltpu.make_async_copy(src2.at[s, pl.ds(0, EB)], sib[islot],
                                  semi).wait()
            pltpu.make_async_copy(src2.at[s, pl.ds(0, EB)], dib[islot],
                                  semi).wait()
            for k in range(EB // 16):
                sib[islot][pl.ds(k * 16, 16)] = (
                    sib[islot][pl.ds(k * 16, 16)] + cN)
            pltpu.async_copy(tab.at[sib[islot]], gbuf[gslot], sg[gslot])
            pltpu.async_copy(er.at[dib[islot]], erb[gslot], sg[gslot])

        def wait_gathers(gslot):
            pltpu.make_async_copy(tab.at[pl.ds(0, EB)], gbuf[gslot],
                                  sg[gslot]).wait()
            pltpu.make_async_copy(er.at[pl.ds(0, EB)], erb[gslot],
                                  sg[gslot]).wait()

        def wait_scatter(gslot):
            pltpu.make_async_copy(gbuf[gslot], acc.at[pl.ds(0, EB)],
                                  ss[gslot]).wait()

        def process(gslot, islot):
            gb = gbuf[gslot]
            eb = erb[gslot]
            for k in range(EB // 16):
                rows16 = lax.iota(jnp.int32, 16) + k * 16
                for g in range(GL):
                    gi = (c * 128 + g * CW) // GW
                    colv = jnp.full((16,), 128, jnp.int32) + gi
                    el16 = plsc.load_gather(gb, [rows16, colv])
                    er16 = plsc.load_gather(
                        eb, [rows16, jnp.zeros((16,), jnp.int32) + gi])
                    e16 = el16 + er16
                    e16 = jnp.where(e16 > 0, e16, slope * e16)
                    ex16 = jnp.exp(e16)
                    exb[g, pl.ds(k * 16, 16)] = ex16
                    plsc.store_scatter(gb, [rows16, colv], ex16)

            @plsc.parallel_loop(0, EB, step=1, unroll=8)
            def _(e):
                for g in range(GL):
                    exv = plsc.load_gather(
                        exb, [jnp.full((16,), g, jnp.int32),
                              jnp.full((16,), e, jnp.int32)])
                    for cc in range(CW // 16):
                        col = g * CW + cc * 16
                        gb[e, pl.ds(col, 16)] = (
                            gb[e, pl.ds(col, 16)] * exv)
            pltpu.async_copy(gb, acc.at[dib[islot]], ss[gslot], add=True)

        stage_idx(0, 0)
        issue_gathers(0, 0)
        stage_idx(1, 1)

        def batch(b, carry):
            for m in range(6):
                @pl.when(b % 6 == m)
                def _():
                    gslot = m % 3
                    ngslot = (m + 1) % 3
                    wait_gathers(gslot)

                    @pl.when(b >= 2)
                    def _():
                        wait_scatter(ngslot)

                    @pl.when(b + 1 < NB)
                    def _():
                        issue_gathers(ngslot, (m + 1) % 6)
                    process(gslot, m)

                    @pl.when(b + 2 < NB)
                    def _():
                        stage_idx(b + 2, (m + 2) % 6)
            return carry
        lax.fori_loop(0, NB, batch, 0)
        wait_scatter((NB - 2) % 3)
        wait_scatter((NB - 1) % 3)
        plsc.subcore_barrier()

        for jj in range((NCH + 15) // 16):
            j = s + jj * 16

            @pl.when(j < NCH)
            def _():
                off = pl.multiple_of(j * EB, 8)
                ooff = pl.multiple_of(c * N_ + j * EB, 8)
                pltpu.sync_copy(acc.at[pl.ds(off, EB)],
                                out.at[pl.ds(ooff, EB)])

    return sck


_sck_heco = _make_sck(1, 0.01)
_sck_gat = _make_sck(H_, 0.2)


def _tck1_body(dstf, f1, f2, WT0, bT0, WT1, bT1, WT2, bT2, al0, ar0, al1,
               ar1, tab0, tab1, er0, er1):
    f32 = jnp.float32
    dsth = jnp.dot(dstf[...], WT0[...], preferred_element_type=f32) + bT0[...]
    n1 = jnp.dot(f1[...], WT1[...], preferred_element_type=f32) + bT1[...]
    n2 = jnp.dot(f2[...], WT2[...], preferred_element_type=f32) + bT2[...]
    z15 = jnp.zeros((BL, 15), f32)
    hp = jax.lax.Precision.HIGHEST
    el0 = jnp.dot(n1, al0[...], preferred_element_type=f32, precision=hp)
    er0v = jnp.dot(dsth, ar0[...], preferred_element_type=f32, precision=hp)
    el1 = jnp.dot(n2, al1[...], preferred_element_type=f32, precision=hp)
    er1v = jnp.dot(dsth, ar1[...], preferred_element_type=f32, precision=hp)
    tab0[0] = jnp.concatenate([n1[:, :128], el0, z15], 1)
    tab0[1] = jnp.concatenate([n1[:, 128:], el0, z15], 1)
    tab1[0] = jnp.concatenate([n2[:, :128], el1, z15], 1)
    tab1[1] = jnp.concatenate([n2[:, 128:], el1, z15], 1)
    er0[...] = jnp.concatenate([er0v, z15], 1)
    er1[...] = jnp.concatenate([er1v, z15], 1)


def _tck1(dstf, f1, f2, WT0, bT0, WT1, bT1, WT2, bT2, al0, ar0, al1, ar1):
    row = lambda i: (i, 0)
    fix = lambda i: (0, 0)
    f32 = jnp.float32
    return pl.pallas_call(
        _tck1_body,
        grid=(GRID,),
        in_specs=[pl.BlockSpec((BL, 256), row)] * 3
        + [pl.BlockSpec((256, 256), fix), pl.BlockSpec((1, 256), fix)] * 3
        + [pl.BlockSpec((256, 1), fix)] * 4,
        out_specs=[pl.BlockSpec((2, BL, 144), lambda i: (0, i, 0))] * 2
        + [pl.BlockSpec((BL, 16), row)] * 2,
        out_shape=[jax.ShapeDtypeStruct((2, N_, 144), f32)] * 2
        + [jax.ShapeDtypeStruct((N_, 16), f32)] * 2,
    )(dstf, f1, f2, WT0, bT0, WT1, bT1, WT2, bT2, al0, ar0, al1, ar1)


def _elu(x):
    return jnp.where(x > 0, x, jnp.exp(x) - 1.0)


def _tck2a_body(raw0, raw1, W1, b1, W2r, zr0, zr1, wsum):
    @pl.when(pl.program_id(0) == 0)
    def _():
        wsum[...] = jnp.zeros((1, 2), jnp.float32)

    outs = []
    for ref in (raw0, raw1):
        sden = ref[0][:, 128:129] + 1e-9
        o = jnp.concatenate([ref[0][:, :128], ref[1][:, :128]], 1) / sden
        outs.append(_elu(o))
    zr0[...] = outs[0]
    zr1[...] = outs[1]
    sc = []
    for z in outs:
        t = jnp.tanh(jnp.dot(z, W1[...],
                             preferred_element_type=jnp.float32) + b1[...])
        sc.append(jnp.sum(t * W2r[...]))
    wsum[...] += jnp.stack(sc).reshape(1, 2)


def _tck2a(raw0, raw1, W1, b1, W2r):
    fix = lambda i: (0, 0)
    row = lambda i: (i, 0)
    f32 = jnp.float32
    return pl.pallas_call(
        _tck2a_body,
        grid=(GRID,),
        in_specs=[pl.BlockSpec((2, BL, 144), lambda i: (0, i, 0))] * 2
        + [pl.BlockSpec((256, 128), fix), pl.BlockSpec((1, 128), fix),
           pl.BlockSpec((1, 128), fix)],
        out_specs=[pl.BlockSpec((BL, 256), row)] * 2
        + [pl.BlockSpec((1, 2), fix)],
        out_shape=[jax.ShapeDtypeStruct((N_, 256), f32)] * 2
        + [jax.ShapeDtypeStruct((1, 2), f32)],
    )(raw0, raw1, W1, b1, W2r)


def _tck2b_body(zr0, zr1, wsum, gW0, galf0, garf0, gW1, galf1, garf1,
                gtab0, gtab1, erg0, erg1):
    f32 = jnp.float32
    w = wsum[...] * (1.0 / N_)
    m = jnp.max(w)
    ew = jnp.exp(w - m)
    a = ew / jnp.sum(ew)
    z = a[0, 0] * zr0[...] + a[0, 1] * zr1[...]
    z8 = jnp.zeros((BL, 8), f32)
    for gW, galf, garf, gtab, erg in ((gW0, galf0, garf0, gtab0, erg0),
                                      (gW1, galf1, garf1, gtab1, erg1)):
        h = jnp.dot(z, gW[...], preferred_element_type=f32)
        hp = jax.lax.Precision.HIGHEST
        el = jnp.dot(h, galf[...], preferred_element_type=f32, precision=hp)
        er = jnp.dot(h, garf[...], preferred_element_type=f32, precision=hp)
        gtab[0] = jnp.concatenate([h[:, :128], el, z8], 1)
        gtab[1] = jnp.concatenate([h[:, 128:], el, z8], 1)
        erg[...] = jnp.concatenate([er, z8], 1)


def _tck2b(zr0, zr1, wsum, gW0, galf0, garf0, gW1, galf1, garf1):
    fix = lambda i: (0, 0)
    row = lambda i: (i, 0)
    f32 = jnp.float32
    return pl.pallas_call(
        _tck2b_body,
        grid=(GRID,),
        in_specs=[pl.BlockSpec((BL, 256), row)] * 2
        + [pl.BlockSpec((1, 2), fix)]
        + [pl.BlockSpec((256, 256), fix), pl.BlockSpec((256, 8), fix),
           pl.BlockSpec((256, 8), fix)] * 2,
        out_specs=[pl.BlockSpec((2, BL, 144), lambda i: (0, i, 0))] * 2
        + [pl.BlockSpec((BL, 16), row)] * 2,
        out_shape=[jax.ShapeDtypeStruct((2, N_, 144), f32)] * 2
        + [jax.ShapeDtypeStruct((N_, 16), f32)] * 2,
    )(zr0, zr1, wsum, gW0, galf0, garf0, gW1, galf1, garf1)


def _tck3a_body(rawg0, rawg1, gb0f, gb1f, W1, b1, W2r, z1, z2, wsum):
    @pl.when(pl.program_id(0) == 0)
    def _():
        wsum[...] = jnp.zeros((1, 2), jnp.float32)

    outs = []
    for ref, gbf in ((rawg0, gb0f), (rawg1, gb1f)):
        s = jnp.concatenate([ref[0][:, 128:132], ref[1][:, 132:136]], 1)
        raw = jnp.concatenate([ref[0][:, :128], ref[1][:, :128]], 1)
        cols = []
        for h in range(H_):
            y = (raw[:, h * 32:(h + 1) * 32] / (s[:, h:h + 1] + 1e-9)
                 + gbf[:, h * 32:(h + 1) * 32])
            cols.append(_elu(y))
        outs.append(jnp.concatenate(cols, 1))
    z1[...] = outs[0]
    z2[...] = outs[1]
    sc = []
    for z in outs:
        t = jnp.tanh(jnp.dot(z, W1[...],
                             preferred_element_type=jnp.float32) + b1[...])
        sc.append(jnp.sum(t * W2r[...]))
    wsum[...] += jnp.stack(sc).reshape(1, 2)


def _tck3a(rawg0, rawg1, gb0f, gb1f, W1, b1, W2r):
    fix = lambda i: (0, 0)
    row = lambda i: (i, 0)
    f32 = jnp.float32
    return pl.pallas_call(
        _tck3a_body,
        grid=(GRID,),
        in_specs=[pl.BlockSpec((2, BL, 144), lambda i: (0, i, 0))] * 2
        + [pl.BlockSpec((1, 256), fix)] * 2
        + [pl.BlockSpec((256, 128), fix), pl.BlockSpec((1, 128), fix),
           pl.BlockSpec((1, 128), fix)],
        out_specs=[pl.BlockSpec((BL, 256), row)] * 2
        + [pl.BlockSpec((1, 2), fix)],
        out_shape=[jax.ShapeDtypeStruct((N_, 256), f32)] * 2
        + [jax.ShapeDtypeStruct((1, 2), f32)],
    )(rawg0, rawg1, gb0f, gb1f, W1, b1, W2r)


def _tck3b_body(z1, z2, wsum, out):
    w = wsum[...] * (1.0 / N_)
    m = jnp.max(w)
    ew = jnp.exp(w - m)
    a = ew / jnp.sum(ew)
    out[...] = a[0, 0] * z1[...] + a[0, 1] * z2[...]


def _tck3b(z1, z2, wsum):
    fix = lambda i: (0, 0)
    row = lambda i: (i, 0)
    return pl.pallas_call(
        _tck3b_body,
        grid=(GRID,),
        in_specs=[pl.BlockSpec((BL, 256), row)] * 2
        + [pl.BlockSpec((1, 2), fix)],
        out_specs=pl.BlockSpec((BL, 256), row),
        out_shape=jax.ShapeDtypeStruct((N_, 256), jnp.float32),
    )(z1, z2, wsum)


def kernel(dst_feat, feat0, feat1, feat2, WT0, bT0, WT1, bT1, WT2, bT2,
           srn_al0, srn_ar0, srn_al1, srn_ar1, gW0, gal0, gar0, gb0, gW1,
           gal1, gar1, gb1, sa1_W1, sa1_b1, sa1_W2, sa2_W1, sa2_b1, sa2_W2,
           sc_edge0, sc_edge1, mp_edge0, mp_edge1):
    r1 = lambda v: v.reshape(1, -1)
    rc = lambda v: v.reshape(-1, 1)
    tab0, tab1, er0, er1 = _tck1(
        dst_feat, feat1, feat2, WT0, r1(bT0), WT1, r1(bT1), WT2, r1(bT2),
        rc(srn_al0), rc(srn_ar0), rc(srn_al1), rc(srn_ar1))

    e2 = lambda e: (e[0].reshape(16, E_ // 16), e[1].reshape(16, E_ // 16))
    s0, dl0 = e2(sc_edge0)
    s1, dl1 = e2(sc_edge1)
    raw0 = _sck_heco(tab0.reshape(2 * N_, 144), er0, s0, dl0)
    raw1 = _sck_heco(tab1.reshape(2 * N_, 144), er1, s1, dl1)

    zr0, zr1, wsum = _tck2a(raw0.reshape(2, N_, 144), raw1.reshape(2, N_, 144),
                            sa1_W1, r1(sa1_b1), r1(sa1_W2))
    hmask = jnp.repeat(jnp.eye(H_, dtype=jnp.float32), DH_, axis=0)
    bd = lambda v: hmask * v.reshape(-1, 1)
    gtab0, gtab1, erg0, erg1 = _tck2b(
        zr0, zr1, wsum, gW0, bd(gal0), bd(gar0), gW1, bd(gal1), bd(gar1))

    sg0, dgl0 = e2(mp_edge0)
    sg1, dgl1 = e2(mp_edge1)
    rawg0 = _sck_gat(gtab0.reshape(2 * N_, 144), erg0, sg0, dgl0)
    rawg1 = _sck_gat(gtab1.reshape(2 * N_, 144), erg1, sg1, dgl1)

    z1, z2, wsum2 = _tck3a(rawg0.reshape(2, N_, 144),
                           rawg1.reshape(2, N_, 144),
                           r1(gb0), r1(gb1), sa2_W1, r1(sa2_b1), r1(sa2_W2))
    return _tck3b(z1, z2, wsum2)


# speedup vs baseline: 1.0834x; 1.0834x over previous
import functools

import jax
import jax.numpy as jnp
from jax import lax
from jax.experimental import pallas as pl
from jax.experimental.pallas import tpu as pltpu
from jax.experimental.pallas import tpu_sc as plsc

N_ = 10000
E_ = 160000
H_ = 8
DH_ = 32
BL = 1000
GRID = N_ // BL
EB = 80
NB = E_ // 16 // EB
RPW = N_ // 16


def _make_sck(G, slope):
    GW = 256 // G
    GL = max(1, 128 // GW)
    CW = 128 // GL
    mesh = plsc.VectorSubcoreMesh(core_axis_name="c", subcore_axis_name="s")

    @functools.partial(
        pl.kernel,
        out_type=jax.ShapeDtypeStruct((2 * N_, 144), jnp.float32),
        mesh=mesh,
        compiler_params=pltpu.CompilerParams(use_tc_tiling_on_sc=False,
                                             needs_layout_passes=False),
        scratch_types=(
            [pltpu.VMEM((EB,), jnp.int32)] * 12
            + [pltpu.VMEM((EB, 144), jnp.float32)] * 3
            + [pltpu.VMEM((EB, 16), jnp.float32)] * 3
            + [
                pltpu.VMEM((16, EB), jnp.float32),
                pltpu.VMEM_SHARED((N_, 144), jnp.float32),
            ]
            + [pltpu.SemaphoreType.DMA] * 7
        ),
    )
    def sck(tab, er, src2, dst2, out, si0, si1, si2, si3, si4, si5,
            di0, di1, di2, di3, di4, di5, gb0, gb1, gb2, eb0, eb1, eb2,
            exb, acc, semi, sg0, sg1, sg2, ss0, ss1, ss2):
        c = lax.axis_index("c")
        s = lax.axis_index("s")
        zero16 = jnp.zeros((16,), jnp.float32)
        sib = (si0, si1, si2, si3, si4, si5)
        dib = (di0, di1, di2, di3, di4, di5)
        gbuf = (gb0, gb1, gb2)
        erb = (eb0, eb1, eb2)
        sg = (sg0, sg1, sg2)
        ss = (ss0, ss1, ss2)

        def zrow(e, carry):
            for cc in range(9):
                gb0[e, pl.ds(cc * 16, 16)] = zero16
            return carry
        lax.fori_loop(0, EB, zrow, 0)
        NCH = N_ // EB
        for jj in range((NCH + 15) // 16):
            j = s + jj * 16

            @pl.when(j < NCH)
            def _():
                off = pl.multiple_of(j * EB, 8)
                pltpu.sync_copy(gb0, acc.at[pl.ds(off, EB)])

        cN = c * N_
        plsc.subcore_barrier()

        def stage_idx(b, islot):
            boff = pl.multiple_of(b * EB, 8)
            pltpu.async_copy(src2.at[s, pl.ds(boff, EB)], sib[islot], semi)
            pltpu.async_copy(dst2.at[s, pl.ds(boff, EB)], dib[islot], semi)

        def issue_gathers(gslot, islot):
            pltpu.make_async_copy(src2.at[s, pl.ds(0, EB)], sib[islot],
                                  semi).wait()
            pltpu.make_async_copy(src2.at[s, pl.ds(0, EB)], dib[islot],
                                  semi).wait()
            for k in range(EB // 16):
                sib[islot][pl.ds(k * 16, 16)] = (
                    sib[islot][pl.ds(k * 16, 16)] + cN)
            pltpu.async_copy(tab.at[sib[islot]], gbuf[gslot], sg[gslot])
            pltpu.async_copy(er.at[dib[islot]], erb[gslot], sg[gslot])

        def wait_gathers(gslot):
            pltpu.make_async_copy(tab.at[pl.ds(0, EB)], gbuf[gslot],
                                  sg[gslot]).wait()
            pltpu.make_async_copy(er.at[pl.ds(0, EB)], erb[gslot],
                                  sg[gslot]).wait()

        def wait_scatter(gslot):
            pltpu.make_async_copy(gbuf[gslot], acc.at[pl.ds(0, EB)],
                                  ss[gslot]).wait()

        def process(gslot, islot):
            gb = gbuf[gslot]
            eb = erb[gslot]
            for k in range(EB // 16):
                rows16 = lax.iota(jnp.int32, 16) + k * 16
                for g in range(GL):
                    gi = (c * 128 + g * CW) // GW
                    colv = jnp.full((16,), 128, jnp.int32) + gi
                    el16 = plsc.load_gather(gb, [rows16, colv])
                    er16 = plsc.load_gather(
                        eb, [rows16, jnp.zeros((16,), jnp.int32) + gi])
                    e16 = el16 + er16
                    e16 = jnp.where(e16 > 0, e16, slope * e16)
                    ex16 = jnp.exp(e16)
                    exb[g, pl.ds(k * 16, 16)] = ex16
                    plsc.store_scatter(gb, [rows16, colv], ex16)

            @plsc.parallel_loop(0, EB, step=1, unroll=4)
            def _(e):
                for g in range(GL):
                    exv = plsc.load_gather(
                        exb, [jnp.full((16,), g, jnp.int32),
                              jnp.full((16,), e, jnp.int32)])
                    for cc in range(CW // 16):
                        col = g * CW + cc * 16
                        gb[e, pl.ds(col, 16)] = (
                            gb[e, pl.ds(col, 16)] * exv)
            pltpu.async_copy(gb, acc.at[dib[islot]], ss[gslot], add=True)

        stage_idx(0, 0)
        issue_gathers(0, 0)
        stage_idx(1, 1)

        def batch(b, carry):
            for m in range(6):
                @pl.when(b % 6 == m)
                def _():
                    gslot = m % 3
                    ngslot = (m + 1) % 3
                    wait_gathers(gslot)

                    @pl.when(b >= 2)
                    def _():
                        wait_scatter(ngslot)

                    @pl.when(b + 1 < NB)
                    def _():
                        issue_gathers(ngslot, (m + 1) % 6)
                    process(gslot, m)

                    @pl.when(b + 2 < NB)
                    def _():
                        stage_idx(b + 2, (m + 2) % 6)
            return carry
        lax.fori_loop(0, NB, batch, 0)
        wait_scatter((NB - 2) % 3)
        wait_scatter((NB - 1) % 3)
        plsc.subcore_barrier()

        for jj in range((NCH + 15) // 16):
            j = s + jj * 16

            @pl.when(j < NCH)
            def _():
                off = pl.multiple_of(j * EB, 8)
                ooff = pl.multiple_of(c * N_ + j * EB, 8)
                pltpu.sync_copy(acc.at[pl.ds(off, EB)],
                                out.at[pl.ds(ooff, EB)])

    return sck


_sck_heco = _make_sck(1, 0.01)
_sck_gat = _make_sck(H_, 0.2)


def _tck1_body(dstf, f1, f2, WT0, bT0, WT1, bT1, WT2, bT2, al0, ar0, al1,
               ar1, tab0, tab1, er0, er1):
    f32 = jnp.float32
    dsth = jnp.dot(dstf[...], WT0[...], preferred_element_type=f32) + bT0[...]
    n1 = jnp.dot(f1[...], WT1[...], preferred_element_type=f32) + bT1[...]
    n2 = jnp.dot(f2[...], WT2[...], preferred_element_type=f32) + bT2[...]
    z15 = jnp.zeros((BL, 15), f32)
    hp = jax.lax.Precision.HIGHEST
    el0 = jnp.dot(n1, al0[...], preferred_element_type=f32, precision=hp)
    er0v = jnp.dot(dsth, ar0[...], preferred_element_type=f32, precision=hp)
    el1 = jnp.dot(n2, al1[...], preferred_element_type=f32, precision=hp)
    er1v = jnp.dot(dsth, ar1[...], preferred_element_type=f32, precision=hp)
    tab0[0] = jnp.concatenate([n1[:, :128], el0, z15], 1)
    tab0[1] = jnp.concatenate([n1[:, 128:], el0, z15], 1)
    tab1[0] = jnp.concatenate([n2[:, :128], el1, z15], 1)
    tab1[1] = jnp.concatenate([n2[:, 128:], el1, z15], 1)
    er0[...] = jnp.concatenate([er0v, z15], 1)
    er1[...] = jnp.concatenate([er1v, z15], 1)


def _tck1(dstf, f1, f2, WT0, bT0, WT1, bT1, WT2, bT2, al0, ar0, al1, ar1):
    row = lambda i: (i, 0)
    fix = lambda i: (0, 0)
    f32 = jnp.float32
    return pl.pallas_call(
        _tck1_body,
        grid=(GRID,),
        in_specs=[pl.BlockSpec((BL, 256), row)] * 3
        + [pl.BlockSpec((256, 256), fix), pl.BlockSpec((1, 256), fix)] * 3
        + [pl.BlockSpec((256, 1), fix)] * 4,
        out_specs=[pl.BlockSpec((2, BL, 144), lambda i: (0, i, 0))] * 2
        + [pl.BlockSpec((BL, 16), row)] * 2,
        out_shape=[jax.ShapeDtypeStruct((2, N_, 144), f32)] * 2
        + [jax.ShapeDtypeStruct((N_, 16), f32)] * 2,
    )(dstf, f1, f2, WT0, bT0, WT1, bT1, WT2, bT2, al0, ar0, al1, ar1)


def _elu(x):
    return jnp.where(x > 0, x, jnp.exp(x) - 1.0)


def _tck2a_body(raw0, raw1, W1, b1, W2r, zr0, zr1, wsum):
    @pl.when(pl.program_id(0) == 0)
    def _():
        wsum[...] = jnp.zeros((1, 2), jnp.float32)

    outs = []
    for ref in (raw0, raw1):
        sden = ref[0][:, 128:129] + 1e-9
        o = jnp.concatenate([ref[0][:, :128], ref[1][:, :128]], 1) / sden
        outs.append(_elu(o))
    zr0[...] = outs[0]
    zr1[...] = outs[1]
    sc = []
    for z in outs:
        t = jnp.tanh(jnp.dot(z, W1[...],
                             preferred_element_type=jnp.float32) + b1[...])
        sc.append(jnp.sum(t * W2r[...]))
    wsum[...] += jnp.stack(sc).reshape(1, 2)


def _tck2a(raw0, raw1, W1, b1, W2r):
    fix = lambda i: (0, 0)
    row = lambda i: (i, 0)
    f32 = jnp.float32
    return pl.pallas_call(
        _tck2a_body,
        grid=(GRID,),
        in_specs=[pl.BlockSpec((2, BL, 144), lambda i: (0, i, 0))] * 2
        + [pl.BlockSpec((256, 128), fix), pl.BlockSpec((1, 128), fix),
           pl.BlockSpec((1, 128), fix)],
        out_specs=[pl.BlockSpec((BL, 256), row)] * 2
        + [pl.BlockSpec((1, 2), fix)],
        out_shape=[jax.ShapeDtypeStruct((N_, 256), f32)] * 2
        + [jax.ShapeDtypeStruct((1, 2), f32)],
    )(raw0, raw1, W1, b1, W2r)


def _tck2b_body(zr0, zr1, wsum, gW0, galf0, garf0, gW1, galf1, garf1,
                gtab0, gtab1, erg0, erg1):
    f32 = jnp.float32
    w = wsum[...] * (1.0 / N_)
    m = jnp.max(w)
    ew = jnp.exp(w - m)
    a = ew / jnp.sum(ew)
    z = a[0, 0] * zr0[...] + a[0, 1] * zr1[...]
    z8 = jnp.zeros((BL, 8), f32)
    for gW, galf, garf, gtab, erg in ((gW0, galf0, garf0, gtab0, erg0),
                                      (gW1, galf1, garf1, gtab1, erg1)):
        h = jnp.dot(z, gW[...], preferred_element_type=f32)
        hp = jax.lax.Precision.HIGHEST
        el = jnp.dot(h, galf[...], preferred_element_type=f32, precision=hp)
        er = jnp.dot(h, garf[...], preferred_element_type=f32, precision=hp)
        gtab[0] = jnp.concatenate([h[:, :128], el, z8], 1)
        gtab[1] = jnp.concatenate([h[:, 128:], el, z8], 1)
        erg[...] = jnp.concatenate([er, z8], 1)


def _tck2b(zr0, zr1, wsum, gW0, galf0, garf0, gW1, galf1, garf1):
    fix = lambda i: (0, 0)
    row = lambda i: (i, 0)
    f32 = jnp.float32
    return pl.pallas_call(
        _tck2b_body,
        grid=(GRID,),
        in_specs=[pl.BlockSpec((BL, 256), row)] * 2
        + [pl.BlockSpec((1, 2), fix)]
        + [pl.BlockSpec((256, 256), fix), pl.BlockSpec((256, 8), fix),
           pl.BlockSpec((256, 8), fix)] * 2,
        out_specs=[pl.BlockSpec((2, BL, 144), lambda i: (0, i, 0))] * 2
        + [pl.BlockSpec((BL, 16), row)] * 2,
        out_shape=[jax.ShapeDtypeStruct((2, N_, 144), f32)] * 2
        + [jax.ShapeDtypeStruct((N_, 16), f32)] * 2,
    )(zr0, zr1, wsum, gW0, galf0, garf0, gW1, galf1, garf1)


def _tck3a_body(rawg0, rawg1, gb0f, gb1f, W1, b1, W2r, z1, z2, wsum):
    @pl.when(pl.program_id(0) == 0)
    def _():
        wsum[...] = jnp.zeros((1, 2), jnp.float32)

    outs = []
    for ref, gbf in ((rawg0, gb0f), (rawg1, gb1f)):
        s = jnp.concatenate([ref[0][:, 128:132], ref[1][:, 132:136]], 1)
        raw = jnp.concatenate([ref[0][:, :128], ref[1][:, :128]], 1)
        cols = []
        for h in range(H_):
            y = (raw[:, h * 32:(h + 1) * 32] / (s[:, h:h + 1] + 1e-9)
                 + gbf[:, h * 32:(h + 1) * 32])
            cols.append(_elu(y))
        outs.append(jnp.concatenate(cols, 1))
    z1[...] = outs[0]
    z2[...] = outs[1]
    sc = []
    for z in outs:
        t = jnp.tanh(jnp.dot(z, W1[...],
                             preferred_element_type=jnp.float32) + b1[...])
        sc.append(jnp.sum(t * W2r[...]))
    wsum[...] += jnp.stack(sc).reshape(1, 2)


def _tck3a(rawg0, rawg1, gb0f, gb1f, W1, b1, W2r):
    fix = lambda i: (0, 0)
    row = lambda i: (i, 0)
    f32 = jnp.float32
    return pl.pallas_call(
        _tck3a_body,
        grid=(GRID,),
        in_specs=[pl.BlockSpec((2, BL, 144), lambda i: (0, i, 0))] * 2
        + [pl.BlockSpec((1, 256), fix)] * 2
        + [pl.BlockSpec((256, 128), fix), pl.BlockSpec((1, 128), fix),
           pl.BlockSpec((1, 128), fix)],
        out_specs=[pl.BlockSpec((BL, 256), row)] * 2
        + [pl.BlockSpec((1, 2), fix)],
        out_shape=[jax.ShapeDtypeStruct((N_, 256), f32)] * 2
        + [jax.ShapeDtypeStruct((1, 2), f32)],
    )(rawg0, rawg1, gb0f, gb1f, W1, b1, W2r)


def _tck3b_body(z1, z2, wsum, out):
    w = wsum[...] * (1.0 / N_)
    m = jnp.max(w)
    ew = jnp.exp(w - m)
    a = ew / jnp.sum(ew)
    out[...] = a[0, 0] * z1[...] + a[0, 1] * z2[...]


def _tck3b(z1, z2, wsum):
    fix = lambda i: (0, 0)
    row = lambda i: (i, 0)
    return pl.pallas_call(
        _tck3b_body,
        grid=(GRID,),
        in_specs=[pl.BlockSpec((BL, 256), row)] * 2
        + [pl.BlockSpec((1, 2), fix)],
        out_specs=pl.BlockSpec((BL, 256), row),
        out_shape=jax.ShapeDtypeStruct((N_, 256), jnp.float32),
    )(z1, z2, wsum)


def kernel(dst_feat, feat0, feat1, feat2, WT0, bT0, WT1, bT1, WT2, bT2,
           srn_al0, srn_ar0, srn_al1, srn_ar1, gW0, gal0, gar0, gb0, gW1,
           gal1, gar1, gb1, sa1_W1, sa1_b1, sa1_W2, sa2_W1, sa2_b1, sa2_W2,
           sc_edge0, sc_edge1, mp_edge0, mp_edge1):
    r1 = lambda v: v.reshape(1, -1)
    rc = lambda v: v.reshape(-1, 1)
    tab0, tab1, er0, er1 = _tck1(
        dst_feat, feat1, feat2, WT0, r1(bT0), WT1, r1(bT1), WT2, r1(bT2),
        rc(srn_al0), rc(srn_ar0), rc(srn_al1), rc(srn_ar1))

    e2 = lambda e: (e[0].reshape(16, E_ // 16), e[1].reshape(16, E_ // 16))
    s0, dl0 = e2(sc_edge0)
    s1, dl1 = e2(sc_edge1)
    raw0 = _sck_heco(tab0.reshape(2 * N_, 144), er0, s0, dl0)
    raw1 = _sck_heco(tab1.reshape(2 * N_, 144), er1, s1, dl1)

    zr0, zr1, wsum = _tck2a(raw0.reshape(2, N_, 144), raw1.reshape(2, N_, 144),
                            sa1_W1, r1(sa1_b1), r1(sa1_W2))
    hmask = jnp.repeat(jnp.eye(H_, dtype=jnp.float32), DH_, axis=0)
    bd = lambda v: hmask * v.reshape(-1, 1)
    gtab0, gtab1, erg0, erg1 = _tck2b(
        zr0, zr1, wsum, gW0, bd(gal0), bd(gar0), gW1, bd(gal1), bd(gar1))

    sg0, dgl0 = e2(mp_edge0)
    sg1, dgl1 = e2(mp_edge1)
    rawg0 = _sck_gat(gtab0.reshape(2 * N_, 144), erg0, sg0, dgl0)
    rawg1 = _sck_gat(gtab1.reshape(2 * N_, 144), erg1, sg1, dgl1)

    z1, z2, wsum2 = _tck3a(rawg0.reshape(2, N_, 144),
                           rawg1.reshape(2, N_, 144),
                           r1(gb0), r1(gb1), sa2_W1, r1(sa2_b1), r1(sa2_W2))
    return _tck3b(z1, z2, wsum2)
